# baseline (speedup 1.0000x reference)
import functools

import jax
import jax.numpy as jnp
from jax import lax
from jax.experimental import pallas as pl
from jax.experimental.pallas import tpu as pltpu
from jax.experimental.pallas import tpu_sc as plsc

D_MODEL = 64


@functools.lru_cache(maxsize=None)
def _embed_lookup(n_seq: int, n_pos: int):
    info = plsc.get_sparse_core_info()
    NC, NS = info.num_cores, info.num_subcores
    NW = NC * NS
    NB = n_seq // 128
    assert NB == NW and n_pos % 2 == 0
    mesh = plsc.VectorSubcoreMesh(core_axis_name="c", subcore_axis_name="s")

    @functools.partial(
        pl.kernel,
        mesh=mesh,
        out_type=jax.ShapeDtypeStruct((n_pos, 8, NB, 8, 128), jnp.float32),
        scratch_types=[
            pltpu.VMEM((n_pos, 128), jnp.int32),
            pltpu.VMEM((2, 128), jnp.int32),
            pltpu.VMEM((2, 128, 128), jnp.float32),
            pltpu.VMEM((2, 8, 8, 128), jnp.float32),
            pltpu.SemaphoreType.DMA,
            pltpu.SemaphoreType.DMA,
            pltpu.SemaphoreType.DMA,
            pltpu.SemaphoreType.DMA,
        ],
        compiler_params=pltpu.CompilerParams(
            use_tc_tiling_on_sc=False, needs_layout_passes=False
        ),
    )
    def body(tok_hbm, tab_hbm, out_hbm, tokv, idxv, stag, ostg,
             sg0, sg1, so0, so1):
        w = lax.axis_index("s") * NC + lax.axis_index("c")
        pltpu.sync_copy(tok_hbm.at[:, pl.ds(w * 128, 128)], tokv)
        iota = lax.iota(jnp.int32, 16)
        sems_g = (sg0, sg1)
        sems_o = (so0, so1)

        def prep_and_gather(j, b):
            for g in range(8):
                t = tokv[j, pl.ds(16 * g, 16)]
                idxv[b, pl.ds(16 * g, 16)] = lax.shift_right_logical(t, 1)
            pltpu.async_copy(tab_hbm.at[idxv.at[b]], stag.at[b], sems_g[b])

        def transpose(j, b):
            for g in range(8):
                t = tokv[j, pl.ds(16 * g, 16)]
                col0 = lax.bitwise_and(t, 1) * 64
                k = iota + 16 * g
                for q in range(8):
                    for u in range(8):
                        v = plsc.load_gather(stag.at[b], [k, col0 + (8 * q + u)])
                        ostg[b, q, u, pl.ds(16 * g, 16)] = v

        prep_and_gather(0, 0)
        prep_and_gather(1, 1)

        def pair(j2, carry):
            for b in range(2):
                j = 2 * j2 + b
                pltpu.make_async_copy(
                    tab_hbm.at[idxv.at[b]], stag.at[b], sems_g[b]
                ).wait()

                @pl.when(j2 > 0)
                def _():
                    pltpu.make_async_copy(
                        ostg.at[b], out_hbm.at[j, :, w], sems_o[b]
                    ).wait()

                transpose(j, b)
                pltpu.async_copy(ostg.at[b], out_hbm.at[j, :, w], sems_o[b])

                @pl.when(j2 < n_pos // 2 - 1)
                def _():
                    prep_and_gather(j + 2, b)

            return carry

        lax.fori_loop(0, n_pos // 2, pair, 0)
        for b in range(2):
            pltpu.make_async_copy(
                ostg.at[b], out_hbm.at[0, :, w], sems_o[b]
            ).wait()

    return body


def kernel(tokens, W_E):
    n_seq, n_pos = tokens.shape
    tok_t = tokens.T
    tab = W_E.reshape(W_E.shape[0] // 2, 128)
    out5 = _embed_lookup(n_seq, n_pos)(tok_t, tab)
    return out5.transpose(2, 4, 0, 1, 3).reshape(n_seq, n_pos, D_MODEL)


# speedup vs baseline: 1.4315x; 1.4315x over previous
import functools

import jax
import jax.numpy as jnp
from jax import lax
from jax.experimental import pallas as pl
from jax.experimental.pallas import tpu as pltpu
from jax.experimental.pallas import tpu_sc as plsc

D_MODEL = 64


@functools.lru_cache(maxsize=None)
def _embed_lookup(n_seq: int, n_pos: int):
    info = plsc.get_sparse_core_info()
    NC, NS = info.num_cores, info.num_subcores
    NW = NC * NS
    NB = n_seq // 128
    assert NB == NW and n_pos % 2 == 0
    mesh = plsc.VectorSubcoreMesh(core_axis_name="c", subcore_axis_name="s")

    @functools.partial(
        pl.kernel,
        mesh=mesh,
        out_type=jax.ShapeDtypeStruct((n_pos, 8, NB, 8, 128), jnp.float32),
        scratch_types=[
            pltpu.VMEM((n_pos, 128), jnp.int32),
            pltpu.VMEM((2, 128), jnp.int32),
            pltpu.VMEM((2, 128, 128), jnp.float32),
            pltpu.VMEM((2, 8, 8, 128), jnp.float32),
            pltpu.SemaphoreType.DMA,
            pltpu.SemaphoreType.DMA,
            pltpu.SemaphoreType.DMA,
            pltpu.SemaphoreType.DMA,
        ],
        compiler_params=pltpu.CompilerParams(
            use_tc_tiling_on_sc=True, needs_layout_passes=False
        ),
    )
    def body(tok_hbm, tab_hbm, out_hbm, tokv, idxv, stag, ostg,
             sg0, sg1, so0, so1):
        w = lax.axis_index("s") * NC + lax.axis_index("c")
        pltpu.sync_copy(tok_hbm.at[:, pl.ds(w * 128, 128)], tokv)
        iota = lax.iota(jnp.int32, 16)
        sems_g = (sg0, sg1)
        sems_o = (so0, so1)

        def prep_and_gather(j, b):
            for g in range(8):
                t = tokv[j, pl.ds(16 * g, 16)]
                idxv[b, pl.ds(16 * g, 16)] = lax.shift_right_logical(t, 1)
            pltpu.async_copy(tab_hbm.at[idxv.at[b]], stag.at[b], sems_g[b])

        def transpose(j, b):
            for g in range(8):
                t = tokv[j, pl.ds(16 * g, 16)]
                col0 = lax.bitwise_and(t, 1) * 64
                k = iota + 16 * g
                for q in range(8):
                    vs = [
                        plsc.load_gather(stag.at[b], [k, col0 + (8 * q + u)])
                        for u in range(8)
                    ]
                    for u in range(8):
                        ostg[b, q, u, pl.ds(16 * g, 16)] = vs[u]

        prep_and_gather(0, 0)
        prep_and_gather(1, 1)

        def pair(j2, carry):
            for b in range(2):
                j = 2 * j2 + b
                pltpu.make_async_copy(
                    tab_hbm.at[idxv.at[b]], stag.at[b], sems_g[b]
                ).wait()

                @pl.when(j2 > 0)
                def _():
                    pltpu.make_async_copy(
                        ostg.at[b], out_hbm.at[j, :, w], sems_o[b]
                    ).wait()

                transpose(j, b)
                pltpu.async_copy(ostg.at[b], out_hbm.at[j, :, w], sems_o[b])

                @pl.when(j2 < n_pos // 2 - 1)
                def _():
                    prep_and_gather(j + 2, b)

            return carry

        lax.fori_loop(0, n_pos // 2, pair, 0)
        for b in range(2):
            pltpu.make_async_copy(
                ostg.at[b], out_hbm.at[0, :, w], sems_o[b]
            ).wait()

    return body


def kernel(tokens, W_E):
    n_seq, n_pos = tokens.shape
    tok_t = tokens.T
    tab = W_E.reshape(W_E.shape[0] // 2, 128)
    out5 = _embed_lookup(n_seq, n_pos)(tok_t, tab)
    return out5.transpose(2, 4, 0, 1, 3).reshape(n_seq, n_pos, D_MODEL)


# speedup vs baseline: 1.5120x; 1.0562x over previous
import functools

import jax
import jax.numpy as jnp
from jax import lax
from jax.experimental import pallas as pl
from jax.experimental.pallas import tpu as pltpu
from jax.experimental.pallas import tpu_sc as plsc

D_MODEL = 64


@functools.lru_cache(maxsize=None)
def _embed_lookup(n_seq: int, n_pos: int):
    info = plsc.get_sparse_core_info()
    NC, NS = info.num_cores, info.num_subcores
    NW = NC * NS
    NB = n_seq // 128
    assert NB == NW and n_pos % 2 == 0
    mesh = plsc.VectorSubcoreMesh(core_axis_name="c", subcore_axis_name="s")

    @functools.partial(
        pl.kernel,
        mesh=mesh,
        out_type=jax.ShapeDtypeStruct((n_pos, 8, NB, 8, 128), jnp.float32),
        scratch_types=[
            pltpu.VMEM((4, 128), jnp.int32),
            pltpu.VMEM((2, 128), jnp.int32),
            pltpu.VMEM((2, 128, 128), jnp.float32),
            pltpu.VMEM((2, 8, 8, 128), jnp.float32),
            pltpu.SemaphoreType.DMA,
            pltpu.SemaphoreType.DMA,
            pltpu.SemaphoreType.DMA,
            pltpu.SemaphoreType.DMA,
            pltpu.SemaphoreType.DMA,
        ],
        compiler_params=pltpu.CompilerParams(
            use_tc_tiling_on_sc=True, needs_layout_passes=False
        ),
    )
    def body(tok_hbm, tab_hbm, out_hbm, tokv, idxv, stag, ostg,
             st, sg0, sg1, so0, so1):
        w = lax.axis_index("s") * NC + lax.axis_index("c")
        iota = lax.iota(jnp.int32, 16)
        sems_g = (sg0, sg1)
        sems_o = (so0, so1)

        def tok_fetch(j):
            pltpu.async_copy(
                tok_hbm.at[j, pl.ds(w * 128, 128)], tokv.at[j % 4], st
            )

        def tok_wait(j):
            pltpu.make_async_copy(
                tok_hbm.at[j, pl.ds(w * 128, 128)], tokv.at[j % 4], st
            ).wait()

        def prep_and_gather(j, b):
            for g in range(8):
                t = tokv[j % 4, pl.ds(16 * g, 16)]
                idxv[b, pl.ds(16 * g, 16)] = lax.shift_right_logical(t, 1)
            pltpu.async_copy(
                tab_hbm.at[idxv.at[b]],
                stag.at[b, :, pl.ds(0, 128)],
                sems_g[b],
            )

        def transpose(j, b):
            for g in range(8):
                t = tokv[j % 4, pl.ds(16 * g, 16)]
                col0 = lax.bitwise_and(t, 1) * 64
                k = iota + 16 * g
                for c00 in range(0, 64, 8):
                    diag = []
                    for d in range(8):
                        cvec = lax.bitwise_and(iota + (c00 + d), 63)
                        v = plsc.load_gather(stag.at[b], [k, col0 + cvec])
                        diag.append((cvec, v))
                    for cvec, v in diag:
                        plsc.store_scatter(
                            ostg.at[b],
                            [
                                lax.shift_right_logical(cvec, 3),
                                lax.bitwise_and(cvec, 7),
                                k,
                            ],
                            v,
                        )

        for j in range(4):
            tok_fetch(j)
        for b in range(2):
            tok_wait(b)
            prep_and_gather(b, b)

        def pair(j2, carry):
            for b in range(2):
                j = 2 * j2 + b
                pltpu.make_async_copy(
                    tab_hbm.at[idxv.at[b]],
                    stag.at[b, :, pl.ds(0, 128)],
                    sems_g[b],
                ).wait()

                @pl.when(j2 > 0)
                def _():
                    pltpu.make_async_copy(
                        ostg.at[b], out_hbm.at[j, :, w], sems_o[b]
                    ).wait()

                transpose(j, b)
                pltpu.async_copy(ostg.at[b], out_hbm.at[j, :, w], sems_o[b])

                @pl.when(j2 < n_pos // 2 - 2)
                def _():
                    tok_fetch(j + 4)

                @pl.when(j2 < n_pos // 2 - 1)
                def _():
                    tok_wait(j + 2)
                    prep_and_gather(j + 2, b)

            return carry

        lax.fori_loop(0, n_pos // 2, pair, 0)
        for b in range(2):
            pltpu.make_async_copy(
                ostg.at[b], out_hbm.at[0, :, w], sems_o[b]
            ).wait()

    return body


def kernel(tokens, W_E):
    n_seq, n_pos = tokens.shape
    tok_t = tokens.T
    tab = W_E.reshape(W_E.shape[0] // 2, 128)
    out5 = _embed_lookup(n_seq, n_pos)(tok_t, tab)
    return out5.transpose(2, 4, 0, 1, 3).reshape(n_seq, n_pos, D_MODEL)


# speedup vs baseline: 1.5837x; 1.0474x over previous
import functools

import jax
import jax.numpy as jnp
from jax import lax
from jax.experimental import pallas as pl
from jax.experimental.pallas import tpu as pltpu
from jax.experimental.pallas import tpu_sc as plsc

D_MODEL = 64


@functools.lru_cache(maxsize=None)
def _embed_lookup(B: int, C: int = 256):
    info = plsc.get_sparse_core_info()
    NC, NS = info.num_cores, info.num_subcores
    NW = NC * NS
    assert B % (8 * NW) == 0
    b_per_w = B // NW
    assert b_per_w % (4 * C) == 0
    n_chunks = b_per_w // C
    n_quads = n_chunks // 4
    mesh = plsc.VectorSubcoreMesh(core_axis_name="c", subcore_axis_name="s")

    @functools.partial(
        pl.kernel,
        mesh=mesh,
        out_type=jax.ShapeDtypeStruct((B, D_MODEL), jnp.float32),
        scratch_types=[
            pltpu.VMEM((b_per_w,), jnp.int32),
            pltpu.VMEM((4, C, D_MODEL), jnp.float32),
            pltpu.SemaphoreType.DMA,
            pltpu.SemaphoreType.DMA,
            pltpu.SemaphoreType.DMA,
            pltpu.SemaphoreType.DMA,
            pltpu.SemaphoreType.DMA,
            pltpu.SemaphoreType.DMA,
            pltpu.SemaphoreType.DMA,
            pltpu.SemaphoreType.DMA,
        ],
        compiler_params=pltpu.CompilerParams(use_tc_tiling_on_sc=False),
    )
    def body(idx_hbm, table_hbm, out_hbm, idx_v, rows,
             sg0, sg1, sg2, sg3, so0, so1, so2, so3):
        w = lax.axis_index("s") * NC + lax.axis_index("c")
        base = w * b_per_w
        pltpu.sync_copy(idx_hbm.at[pl.ds(base, b_per_w)], idx_v)
        sems_g = (sg0, sg1, sg2, sg3)
        sems_o = (so0, so1, so2, so3)

        def gather(i, b):
            pltpu.async_copy(
                table_hbm.at[idx_v.at[pl.ds(i * C, C)]], rows.at[b], sems_g[b]
            )

        def gather_wait(i, b):
            pltpu.make_async_copy(
                table_hbm.at[idx_v.at[pl.ds(i * C, C)]], rows.at[b], sems_g[b]
            ).wait()

        def flush(i, b):
            pltpu.async_copy(
                rows.at[b], out_hbm.at[pl.ds(base + i * C, C)], sems_o[b]
            )

        def flush_wait(i, b):
            pltpu.make_async_copy(
                rows.at[b], out_hbm.at[pl.ds(base + i * C, C)], sems_o[b]
            ).wait()

        gather(0, 0)
        gather(1, 1)

        def quad(q, carry):
            for b in range(4):
                i = 4 * q + b
                gather_wait(i, b)
                flush(i, b)
                if b < 2:
                    @pl.when(q > 0)
                    def _():
                        flush_wait(i - 2, (b + 2) % 4)
                        gather(i + 2, (b + 2) % 4)

                    @pl.when(q == 0)
                    def _():
                        gather(i + 2, (b + 2) % 4)
                else:
                    flush_wait(i - 2, (b + 2) % 4)

                    @pl.when(q < n_quads - 1)
                    def _():
                        gather(i + 2, (b + 2) % 4)

            return carry

        lax.fori_loop(0, n_quads, quad, 0)
        flush_wait(n_chunks - 2, 2)
        flush_wait(n_chunks - 1, 3)

    return body


def kernel(tokens, W_E):
    n_seq, n_tok = tokens.shape
    B = n_seq * n_tok
    flat = tokens.reshape(B)
    out = _embed_lookup(B)(flat, W_E)
    return out.reshape(n_seq, n_tok, D_MODEL)
